# baseline (speedup 1.0000x reference)
import functools

import jax
import jax.numpy as jnp
from jax import lax
from jax.experimental import pallas as pl
from jax.experimental.pallas import tpu as pltpu
from jax.experimental.pallas import tpu_sc as plsc

_DIM = 64
_SCALE = 8.0
_LANES = 16
_NC = 2
_NS = 16
_NW = _NC * _NS
_CHUNK = 128
_TCBN = 4096


def _tc_pack(table_t):
    dim, vocab = table_t.shape
    nblk = (vocab + _TCBN - 1) // _TCBN
    rows = nblk * (_TCBN // 2)

    def body(x_ref, out_ref):
        x = x_ref[...]
        a = x[:, : _TCBN // 2]
        b = x[:, _TCBN // 2 :]
        out_ref[...] = jnp.concatenate([a.T, b.T], axis=1) * _SCALE

    return pl.pallas_call(
        body,
        grid=(nblk,),
        in_specs=[pl.BlockSpec((dim, _TCBN), lambda i: (0, i))],
        out_specs=pl.BlockSpec((_TCBN // 2, 2 * dim), lambda i: (i, 0)),
        out_shape=jax.ShapeDtypeStruct((rows, 2 * dim), jnp.float32),
    )(table_t)


def _sc_embed(idx_t, table2):
    seq, batch = idx_t.shape
    kpl = batch // _CHUNK
    nchunks = seq * kpl
    cpw = nchunks // _NW
    mesh = plsc.VectorSubcoreMesh(core_axis_name="c", subcore_axis_name="s")

    @functools.partial(
        pl.kernel,
        mesh=mesh,
        out_type=jax.ShapeDtypeStruct(
            (seq, _DIM // 8, batch // _CHUNK, 8, _CHUNK), jnp.float32
        ),
        compiler_params=pltpu.CompilerParams(
            use_tc_tiling_on_sc=False, needs_layout_passes=False
        ),
        scratch_types=[
            pltpu.VMEM((_CHUNK,), jnp.int32),
            pltpu.VMEM((_CHUNK,), jnp.int32),
            pltpu.VMEM((_CHUNK,), jnp.int32),
            pltpu.VMEM((_CHUNK,), jnp.int32),
            pltpu.VMEM((_CHUNK,), jnp.int32),
            pltpu.VMEM((_CHUNK,), jnp.int32),
            pltpu.VMEM((_CHUNK, 2 * _DIM), jnp.float32),
            pltpu.VMEM((_CHUNK, 2 * _DIM), jnp.float32),
            pltpu.VMEM((_DIM // 8, 8, _CHUNK + 1), jnp.float32),
            pltpu.VMEM((_DIM // 8, 8, _CHUNK + 1), jnp.float32),
            pltpu.SemaphoreType.DMA,
            pltpu.SemaphoreType.DMA,
            pltpu.SemaphoreType.DMA,
            pltpu.SemaphoreType.DMA,
            pltpu.SemaphoreType.DMA,
            pltpu.SemaphoreType.DMA,
            pltpu.SemaphoreType.DMA,
            pltpu.SemaphoreType.DMA,
        ],
    )
    def body(idx_hbm, table_hbm, out_hbm, i0, i1, i2, i3, h0, h1,
             g0, g1, t0, t1, is0, is1, is2, is3, gs0, gs1, ss0, ss1):
        wid = lax.axis_index("s") * _NC + lax.axis_index("c")
        cbase = wid * cpw

        ibufs = (i0, i1, i2, i3)
        isems = (is0, is1, is2, is3)
        hbufs = (h0, h1)
        gbufs = (g0, g1)
        gsems = (gs0, gs1)
        tbufs = (t0, t1)
        ssems = (ss0, ss1)
        iot = lax.iota(jnp.int32, _LANES)

        def lk(j):
            c = cbase + j
            return c // kpl, lax.rem(c, kpl)

        def idx_start(j, islot):
            l, k = lk(j)
            pltpu.async_copy(idx_hbm.at[l, pl.ds(k * _CHUNK, _CHUNK)],
                             ibufs[islot], isems[islot])

        def idx_wait(islot):
            pltpu.make_async_copy(idx_hbm.at[0, pl.ds(0, _CHUNK)],
                                  ibufs[islot], isems[islot]).wait()

        def pair_rows(islot, gslot):
            src = ibufs[islot]
            dst = hbufs[gslot]
            for c in range(_CHUNK // _LANES):
                sl = pl.ds(c * _LANES, _LANES)
                v = src[sl]
                dst[sl] = (
                    lax.shift_left(lax.shift_right_logical(v, 12), 11)
                    + (v & 2047)
                )

        def gather_start(gslot):
            pltpu.async_copy(table_hbm.at[hbufs[gslot]], gbufs[gslot],
                             gsems[gslot])

        def gather_wait(gslot):
            pltpu.make_async_copy(table_hbm.at[hbufs[0]], gbufs[gslot],
                                  gsems[gslot]).wait()

        def store_start(j, gslot):
            l, k = lk(j)
            pltpu.async_copy(tbufs[gslot].at[:, :, pl.ds(0, _CHUNK)],
                             out_hbm.at[l, :, k], ssems[gslot])

        def store_wait(gslot):
            pltpu.make_async_copy(tbufs[gslot].at[:, :, pl.ds(0, _CHUNK)],
                                  out_hbm.at[0, :, 0],
                                  ssems[gslot]).wait()

        def transpose_select(islot, gslot):
            g = gbufs[gslot]
            t = tbufs[gslot]
            raw = ibufs[islot]

            @plsc.parallel_loop(0, _CHUNK // _LANES, unroll=2)
            def _(kk):
                k0 = kk * _LANES
                offs = (
                    lax.shift_right_logical(raw[pl.ds(k0, _LANES)], 11) & 1
                ) * _DIM
                for ll in range(_LANES):
                    k = k0 + ll
                    ks = iot * 0 + k
                    h = offs[ll]
                    for c in range(_DIM // _LANES):
                        f16 = iot + (c * _LANES)
                        trs = lax.shift_right_logical(f16, 3)
                        frs = f16 & 7
                        v = g[k, pl.ds(h + c * _LANES, _LANES)]
                        plsc.store_scatter(t, [trs, frs, ks], v)

        for j in range(4):
            idx_start(j, j)
        idx_wait(0)
        pair_rows(0, 0)
        gather_start(0)
        idx_wait(1)
        pair_rows(1, 1)
        gather_start(1)

        def step(tt, carry):
            for u in range(4):
                j = 4 * tt + u
                gslot = u % 2

                @pl.when(j >= 2)
                def _():
                    store_wait(gslot)

                gather_wait(gslot)
                transpose_select(u, gslot)

                @pl.when(j + 4 < cpw)
                def _():
                    idx_start(j + 4, u)

                store_start(j, gslot)

                @pl.when(j + 2 < cpw)
                def _():
                    idx_wait((u + 2) % 4)
                    pair_rows((u + 2) % 4, gslot)
                    gather_start(gslot)
            return carry

        lax.fori_loop(0, cpw // 4, step, 0)
        store_wait(0)
        store_wait(1)

    return body(idx_t, table2)


def kernel(x, table):
    b, l = x.shape
    idx_t = x.T.astype(jnp.int32)
    table2 = _tc_pack(table.T)
    out5 = _sc_embed(idx_t, table2)
    outp = jnp.transpose(out5, (2, 4, 0, 1, 3))
    return outp.reshape(b, l, _DIM)
